# baseline (speedup 1.0000x reference)
import functools

import jax
import jax.numpy as jnp
from jax.experimental import pallas as pl
from jax.experimental.pallas import tpu as pltpu

_CHUNK = 16


def _bilstm_chunk_kernel(x_ref, m_ref, wih_ref, whh_ref, b_ref, o_ref,
                         h_ref, c_ref, xg_ref, *, chunk, hd):
    d = pl.program_id(0)
    cstep = pl.program_id(1)
    B = x_ref.shape[1]
    D = x_ref.shape[2]

    @pl.when(cstep == 0)
    def _init():
        h_ref[...] = jnp.zeros_like(h_ref)
        c_ref[...] = jnp.zeros_like(c_ref)

    xg = jnp.dot(x_ref[...].reshape(chunk * B, D), wih_ref[0],
                 preferred_element_type=jnp.float32)
    xg_ref[...] = (xg + b_ref[0]).astype(jnp.bfloat16)

    def run_chunk(order):
        h = h_ref[...]
        c = c_ref[...]
        for j in order:
            g = (xg_ref[pl.ds(j * B, B), :].astype(jnp.float32)
                 + jnp.dot(h.astype(jnp.bfloat16), whh_ref[0],
                           preferred_element_type=jnp.float32))
            i_g = jax.nn.sigmoid(g[:, 0 * hd:1 * hd])
            f_g = jax.nn.sigmoid(g[:, 1 * hd:2 * hd])
            g_g = jnp.tanh(g[:, 2 * hd:3 * hd])
            o_g = jax.nn.sigmoid(g[:, 3 * hd:4 * hd])
            c_new = f_g * c + i_g * g_g
            h_new = o_g * jnp.tanh(c_new)
            m = m_ref[j]
            h = h + m * (h_new - h)
            c = c + m * (c_new - c)
            o_ref[j] = (m * h_new).astype(jnp.bfloat16)
        h_ref[...] = h
        c_ref[...] = c

    @pl.when(d == 0)
    def _fwd():
        run_chunk(range(chunk))

    @pl.when(d == 1)
    def _bwd():
        run_chunk(range(chunk - 1, -1, -1))


def kernel(inputs, mask, w_ih_f, w_hh_f, b_ih_f, b_hh_f,
           w_ih_b, w_hh_b, b_ih_b, b_hh_b):
    B, S, D = inputs.shape
    hd = w_hh_f.shape[1]
    out_dtype = inputs.dtype
    C = _CHUNK if S % _CHUNK == 0 else S
    NC = S // C

    x_t = jnp.transpose(inputs, (1, 0, 2)).astype(jnp.bfloat16)
    m_t = jnp.transpose(mask.astype(jnp.bfloat16), (1, 0))[:, :, None]
    wih = jnp.stack([jnp.transpose(w_ih_f), jnp.transpose(w_ih_b)]
                    ).astype(jnp.bfloat16)
    whh = jnp.stack([jnp.transpose(w_hh_f), jnp.transpose(w_hh_b)]
                    ).astype(jnp.bfloat16)
    bias = jnp.stack([b_ih_f + b_hh_f, b_ih_b + b_hh_b]
                     ).astype(jnp.float32)[:, None, :]

    def chunk_sel(d, c):
        return jnp.where(d == 0, c, NC - 1 - c)

    body = functools.partial(_bilstm_chunk_kernel, chunk=C, hd=hd)

    out_t = pl.pallas_call(
        body,
        out_shape=jax.ShapeDtypeStruct((S, B, 2 * hd), jnp.bfloat16),
        grid_spec=pltpu.PrefetchScalarGridSpec(
            num_scalar_prefetch=0,
            grid=(2, NC),
            in_specs=[
                pl.BlockSpec((C, B, D), lambda d, c: (chunk_sel(d, c), 0, 0)),
                pl.BlockSpec((C, B, 1), lambda d, c: (chunk_sel(d, c), 0, 0)),
                pl.BlockSpec((1, D, 4 * hd), lambda d, c: (d, 0, 0)),
                pl.BlockSpec((1, hd, 4 * hd), lambda d, c: (d, 0, 0)),
                pl.BlockSpec((1, 1, 4 * hd), lambda d, c: (d, 0, 0)),
            ],
            out_specs=pl.BlockSpec((C, B, hd),
                                   lambda d, c: (chunk_sel(d, c), 0, d)),
            scratch_shapes=[
                pltpu.VMEM((B, hd), jnp.float32),
                pltpu.VMEM((B, hd), jnp.float32),
                pltpu.VMEM((C * B, 4 * hd), jnp.bfloat16),
            ],
        ),
        compiler_params=pltpu.CompilerParams(
            dimension_semantics=("parallel", "arbitrary")),
    )(x_t, m_t, wih, whh, bias)

    return jnp.transpose(out_t, (1, 0, 2)).astype(out_dtype)


# speedup vs baseline: 1.1214x; 1.1214x over previous
import functools

import jax
import jax.numpy as jnp
from jax.experimental import pallas as pl
from jax.experimental.pallas import tpu as pltpu

_CHUNK = 32


def _bilstm_chunk_kernel(x_ref, m_ref, wih_ref, whh_ref, b_ref, o_ref,
                         h_ref, c_ref, xg_ref, *, chunk, hd):
    d = pl.program_id(0)
    cstep = pl.program_id(1)
    B = x_ref.shape[1]
    D = x_ref.shape[2]

    @pl.when(cstep == 0)
    def _init():
        h_ref[...] = jnp.zeros_like(h_ref)
        c_ref[...] = jnp.zeros_like(c_ref)

    xg = jnp.dot(x_ref[...].reshape(chunk * B, D), wih_ref[0],
                 preferred_element_type=jnp.float32)
    xg_ref[...] = (xg + b_ref[0]).astype(jnp.bfloat16)

    def run_chunk(order):
        h = h_ref[...]
        c = c_ref[...]
        for j in order:
            g = (xg_ref[pl.ds(j * B, B), :].astype(jnp.float32)
                 + jnp.dot(h.astype(jnp.bfloat16), whh_ref[0],
                           preferred_element_type=jnp.float32))
            i_g = jax.nn.sigmoid(g[:, 0 * hd:1 * hd])
            f_g = jax.nn.sigmoid(g[:, 1 * hd:2 * hd])
            g_g = jnp.tanh(g[:, 2 * hd:3 * hd])
            o_g = jax.nn.sigmoid(g[:, 3 * hd:4 * hd])
            c_new = f_g * c + i_g * g_g
            h_new = o_g * jnp.tanh(c_new)
            m = m_ref[j]
            h = h + m * (h_new - h)
            c = c + m * (c_new - c)
            o_ref[j] = m * h_new
        h_ref[...] = h
        c_ref[...] = c

    @pl.when(d == 0)
    def _fwd():
        run_chunk(range(chunk))

    @pl.when(d == 1)
    def _bwd():
        run_chunk(range(chunk - 1, -1, -1))


def kernel(inputs, mask, w_ih_f, w_hh_f, b_ih_f, b_hh_f,
           w_ih_b, w_hh_b, b_ih_b, b_hh_b):
    B, S, D = inputs.shape
    hd = w_hh_f.shape[1]
    out_dtype = inputs.dtype
    C = _CHUNK if S % _CHUNK == 0 else S
    NC = S // C

    x_t = jnp.transpose(inputs, (1, 0, 2)).astype(jnp.bfloat16)
    m_t = jnp.transpose(mask.astype(jnp.float32), (1, 0))[:, :, None]
    wih = jnp.stack([jnp.transpose(w_ih_f), jnp.transpose(w_ih_b)]
                    ).astype(jnp.bfloat16)
    whh = jnp.stack([jnp.transpose(w_hh_f), jnp.transpose(w_hh_b)]
                    ).astype(jnp.bfloat16)
    bias = jnp.stack([b_ih_f + b_hh_f, b_ih_b + b_hh_b]
                     ).astype(jnp.float32)[:, None, :]

    def chunk_sel(d, c):
        return jnp.where(d == 0, c, NC - 1 - c)

    body = functools.partial(_bilstm_chunk_kernel, chunk=C, hd=hd)

    out_t = pl.pallas_call(
        body,
        out_shape=jax.ShapeDtypeStruct((S, B, 2 * hd), jnp.float32),
        grid_spec=pltpu.PrefetchScalarGridSpec(
            num_scalar_prefetch=0,
            grid=(2, NC),
            in_specs=[
                pl.BlockSpec((C, B, D), lambda d, c: (chunk_sel(d, c), 0, 0)),
                pl.BlockSpec((C, B, 1), lambda d, c: (chunk_sel(d, c), 0, 0)),
                pl.BlockSpec((1, D, 4 * hd), lambda d, c: (d, 0, 0)),
                pl.BlockSpec((1, hd, 4 * hd), lambda d, c: (d, 0, 0)),
                pl.BlockSpec((1, 1, 4 * hd), lambda d, c: (d, 0, 0)),
            ],
            out_specs=pl.BlockSpec((C, B, hd),
                                   lambda d, c: (chunk_sel(d, c), 0, d)),
            scratch_shapes=[
                pltpu.VMEM((B, hd), jnp.float32),
                pltpu.VMEM((B, hd), jnp.float32),
                pltpu.VMEM((C * B, 4 * hd), jnp.bfloat16),
            ],
        ),
        compiler_params=pltpu.CompilerParams(
            dimension_semantics=("parallel", "arbitrary")),
    )(x_t, m_t, wih, whh, bias)

    return jnp.transpose(out_t, (1, 0, 2)).astype(out_dtype)
